# Initial kernel scaffold; baseline (speedup 1.0000x reference)
import jax
import jax.numpy as jnp
from jax.experimental import pallas as pl


def kernel(h, edge_index, r, norm, W1, loop_w1, bias1, W2, loop_w2, bias2):
    raise NotImplementedError("write your pallas kernel here")


# speedup vs baseline: 10.1335x; 10.1335x over previous
import functools

import jax
import jax.numpy as jnp
from jax import lax
from jax.experimental import pallas as pl
from jax.experimental.pallas import tpu as pltpu
from jax.experimental.pallas import tpu_sc as plsc

N = 10000
E = 320000
D = 128
R = 16
B = 8
SUB = D // B

NC = 2
NS = 16
NW = NC * NS
L = 16

C = 128
E_PAD = ((E + NW * C - 1) // (NW * C)) * (NW * C)
EW = E_PAD // NW
NCHUNK = EW // C
ZR = 128
N_PAD = ((N + NS * ZR - 1) // (NS * ZR)) * (NS * ZR)
ROWS_PER_TILE = N_PAD // NS
TILE = 400


def _sc_segment_kernel():
  mesh = plsc.VectorSubcoreMesh(core_axis_name="c", subcore_axis_name="s")

  @functools.partial(
      pl.kernel,
      out_type=jax.ShapeDtypeStruct((NC, N_PAD, D), jnp.float32),
      mesh=mesh,
      compiler_params=pltpu.CompilerParams(needs_layout_passes=False),
      scratch_types=[
          pltpu.VMEM((C,), jnp.int32),
          pltpu.VMEM((C,), jnp.int32),
          pltpu.VMEM((C,), jnp.int32),
          pltpu.VMEM((C,), jnp.float32),
          pltpu.VMEM((C,), jnp.int32),
          pltpu.VMEM((C, D), jnp.float32),
          pltpu.VMEM((ZR, D), jnp.float32),
          pltpu.VMEM_SHARED((N_PAD, D), jnp.float32),
          pltpu.SemaphoreType.DMA,
      ],
  )
  def body(table_hbm, src_hbm, r_hbm, dst_hbm, norm_hbm, out_hbm,
           srcv, rv, dstv, normv, gidxv, rowsv, zbuf, acc, sem):
    cid = lax.axis_index("c")
    sid = lax.axis_index("s")
    wid = sid * NC + cid

    def zrow(i, _):
      for j in range(D // L):
        zbuf[i, pl.ds(j * L, L)] = jnp.zeros((L,), jnp.float32)
      return 0
    lax.fori_loop(0, ZR, zrow, 0)
    for t in range(ROWS_PER_TILE // ZR):
      pltpu.sync_copy(zbuf, acc.at[pl.ds(sid * ROWS_PER_TILE + t * ZR, ZR)])
    plsc.subcore_barrier()

    base = wid * EW

    def chunk(k, _):
      off = base + k * C
      pltpu.sync_copy(src_hbm.at[pl.ds(off, C)], srcv)
      pltpu.sync_copy(r_hbm.at[pl.ds(off, C)], rv)
      pltpu.sync_copy(dst_hbm.at[pl.ds(off, C)], dstv)
      pltpu.sync_copy(norm_hbm.at[pl.ds(off, C)], normv)
      for g in range(C // L):
        sl = pl.ds(g * L, L)
        gidxv[sl] = srcv[sl] * R + rv[sl]
      pltpu.async_copy(table_hbm.at[gidxv], rowsv, sem).wait()

      def edge(e, _):
        nsplat = plsc.load_gather(normv, [jnp.zeros((L,), jnp.int32) + e])
        for j in range(D // L):
          sl = pl.ds(j * L, L)
          rowsv[e, sl] = rowsv[e, sl] * nsplat
        return 0
      lax.fori_loop(0, C, edge, 0)
      pltpu.sync_copy(rowsv, acc.at[dstv], add=True)
      return 0

    lax.fori_loop(0, NCHUNK, chunk, 0)
    plsc.subcore_barrier()

    for t in range(ROWS_PER_TILE // ZR):
      rs = sid * ROWS_PER_TILE + t * ZR
      pltpu.sync_copy(acc.at[pl.ds(rs, ZR)], zbuf)
      pltpu.sync_copy(zbuf, out_hbm.at[cid, pl.ds(rs, ZR)])

  return body


_sc_segment = _sc_segment_kernel()


def _tc_first(h_ref, wflat_ref, lw_ref, b_ref, table_ref, sl_ref):
  ht = h_ref[...]
  table_ref[...] = jnp.dot(ht, wflat_ref[...], preferred_element_type=jnp.float32)
  sl_ref[...] = (jnp.dot(ht, lw_ref[...], preferred_element_type=jnp.float32)
                 + b_ref[...])


def _tc_mid(agg_ref, sl_prev_ref, wflat_ref, lw_ref, b_ref, table_ref, sl_ref):
  ht = jax.nn.relu(agg_ref[0] + agg_ref[1] + sl_prev_ref[...])
  table_ref[...] = jnp.dot(ht, wflat_ref[...], preferred_element_type=jnp.float32)
  sl_ref[...] = (jnp.dot(ht, lw_ref[...], preferred_element_type=jnp.float32)
                 + b_ref[...])


def _tc_last(agg_ref, sl_prev_ref, out_ref):
  out_ref[...] = agg_ref[0] + agg_ref[1] + sl_prev_ref[...]


_ROWS = pl.BlockSpec((TILE, D), lambda i: (i, 0))
_ROWS_RD = pl.BlockSpec((TILE, R * D), lambda i: (i, 0))
_AGG = pl.BlockSpec((NC, TILE, D), lambda i: (0, i, 0))
_WFLAT = pl.BlockSpec((D, R * D), lambda i: (0, 0))
_LW = pl.BlockSpec((D, D), lambda i: (0, 0))
_BIAS = pl.BlockSpec((1, D), lambda i: (0, 0))

_TABLE_OUT = (jax.ShapeDtypeStruct((N, R * D), jnp.float32),
              jax.ShapeDtypeStruct((N, D), jnp.float32))


def _dense_w(W):
  Wm = W.reshape(R, B, SUB, SUB)
  eye = jnp.eye(B, dtype=W.dtype)
  Wd = jnp.einsum('rbij,bc->rbicj', Wm, eye).reshape(R, D, D)
  return Wd.transpose(1, 0, 2).reshape(D, R * D)


def kernel(h, edge_index, r, norm, W1, loop_w1, bias1, W2, loop_w2, bias2):
  src = edge_index[0]
  dst = edge_index[1]
  pad = E_PAD - E
  src_p = jnp.concatenate([src, jnp.zeros((pad,), jnp.int32)])
  dst_p = jnp.concatenate([dst, jnp.zeros((pad,), jnp.int32)])
  r_p = jnp.concatenate([r, jnp.zeros((pad,), jnp.int32)])
  norm_p = jnp.concatenate([norm[:, 0], jnp.zeros((pad,), jnp.float32)])

  wflat1 = _dense_w(W1)
  wflat2 = _dense_w(W2)
  b1 = bias1.reshape(1, D)
  b2 = bias2.reshape(1, D)

  table1, sl1 = pl.pallas_call(
      _tc_first,
      grid=(N // TILE,),
      in_specs=[_ROWS, _WFLAT, _LW, _BIAS],
      out_specs=[_ROWS_RD, _ROWS],
      out_shape=_TABLE_OUT,
  )(h, wflat1, loop_w1, b1)

  agg1 = _sc_segment(table1.reshape(N * R, D), src_p, r_p, dst_p, norm_p)

  table2, sl2 = pl.pallas_call(
      _tc_mid,
      grid=(N // TILE,),
      in_specs=[_AGG, _ROWS, _WFLAT, _LW, _BIAS],
      out_specs=[_ROWS_RD, _ROWS],
      out_shape=_TABLE_OUT,
  )(agg1, sl1, wflat2, loop_w2, b2)

  agg2 = _sc_segment(table2.reshape(N * R, D), src_p, r_p, dst_p, norm_p)

  out = pl.pallas_call(
      _tc_last,
      grid=(N // TILE,),
      in_specs=[_AGG, _ROWS],
      out_specs=_ROWS,
      out_shape=jax.ShapeDtypeStruct((N, D), jnp.float32),
  )(agg2, sl2)
  return out
